# baseline (speedup 1.0000x reference)
import functools

import jax
import jax.numpy as jnp
import numpy as np
from jax import lax
from jax.experimental import pallas as pl
from jax.experimental.pallas import tpu as pltpu
from jax.experimental.pallas import tpu_sc as plsc

VOCAB = 100000
D = 64
MAXLEN = 200
NG1 = 21
MAX_G = 3
BATCH = 4096
SEQLEN = 200
N = BATCH * SEQLEN


def _pe_table():
    pe = np.zeros((MAXLEN, D), dtype=np.float32)
    position = np.arange(MAXLEN, dtype=np.float32)[:, None]
    div_term = np.exp(np.arange(0, D, 2, dtype=np.float32) * (-np.log(10000.0) / D))
    pe[:, 0::2] = np.sin(position * div_term)
    pe[:, 1::2] = np.cos(position * div_term)
    return pe


_PE = _pe_table()


_R = 2000


def _fuse_body(tok_ref, gid_ref, gtab_ref, out_ref):
    gids = gid_ref[...]
    iota = lax.broadcasted_iota(jnp.int32, (_R, NG1), 1)
    counts = jnp.zeros((_R, NG1), jnp.float32)
    for g in range(MAX_G):
        gid_g = lax.slice(gids, (0, g), (_R, g + 1))
        counts = counts + (gid_g == iota).astype(jnp.float32)
    gavg = lax.dot_general(
        counts, gtab_ref[...], (((1,), (0,)), ((), ())),
        preferred_element_type=jnp.float32,
    )
    out_ref[:, :D] = tok_ref[...] + gavg * (1.0 / MAX_G)
    out_ref[:, D:] = jnp.zeros((_R, D), jnp.float32)


def _build_fused(token_table, genre_table, token_to_genres):
    return pl.pallas_call(
        _fuse_body,
        grid=(VOCAB // _R,),
        in_specs=[
            pl.BlockSpec((_R, D), lambda i: (i, 0)),
            pl.BlockSpec((_R, MAX_G), lambda i: (i, 0)),
            pl.BlockSpec((NG1, D), lambda i: (0, 0)),
        ],
        out_specs=pl.BlockSpec((_R, 2 * D), lambda i: (i, 0)),
        out_shape=jax.ShapeDtypeStruct((VOCAB, 2 * D), jnp.float32),
    )(token_table, token_to_genres, genre_table)


_NW = 32
_TPW = N // _NW
_CH = 128
_NCH = _TPW // _CH


def _gather_pe_body(fused_hbm, seqf_hbm, pe_hbm, out_hbm,
                    idx0_v, idx1_v, rows_v, res_v, pe_v, gsems, wsems):
    wid = lax.axis_index("s") * 2 + lax.axis_index("c")
    t00 = wid * _TPW
    idx_refs = (idx0_v, idx1_v)
    pltpu.sync_copy(pe_hbm, pe_v)

    def load_idx(buf, c):
        pltpu.sync_copy(seqf_hbm.at[pl.ds(t00 + c * _CH, _CH)], idx_refs[buf])

    def start_gather(buf, c):
        pltpu.async_copy(
            fused_hbm.at[idx_refs[buf]], rows_v.at[buf], gsems.at[buf])

    def wait_gather(buf):
        pltpu.make_async_copy(
            fused_hbm.at[pl.ds(0, _CH)], rows_v.at[buf], gsems.at[buf]).wait()

    def add_pe(buf, c):
        pb = lax.rem(c * _CH, SEQLEN)

        @pl.loop(0, _CH)
        def _(j):
            for s in range(D // 16):
                sl = pl.ds(s * 16, 16)
                res_v[buf, j, sl] = rows_v[buf, j, sl] + pe_v[pb + j, sl]

    def start_write(buf, c):
        pltpu.async_copy(
            res_v.at[buf], out_hbm.at[pl.ds(t00 + c * _CH, _CH)],
            wsems.at[buf])

    def wait_write(buf):
        pltpu.make_async_copy(
            res_v.at[buf], out_hbm.at[pl.ds(0, _CH)], wsems.at[buf]).wait()

    load_idx(0, 0)
    start_gather(0, 0)
    load_idx(1, 1)
    start_gather(1, 1)

    @pl.loop(0, _NCH - 2, step=2)
    def _(cc):
        for buf in range(2):
            wait_gather(buf)
            add_pe(buf, cc + buf)
            start_write(buf, cc + buf)
        for buf in range(2):
            load_idx(buf, cc + 2 + buf)
            wait_write(buf)
            start_gather(buf, cc + 2 + buf)

    for buf in range(2):
        wait_gather(buf)
        add_pe(buf, _NCH - 2 + buf)
        start_write(buf, _NCH - 2 + buf)
    for buf in range(2):
        wait_write(buf)


@functools.cache
def _gather_pe():
    mesh = plsc.VectorSubcoreMesh(core_axis_name="c", subcore_axis_name="s")
    return pl.kernel(
        _gather_pe_body,
        out_type=jax.ShapeDtypeStruct((N, D), jnp.float32),
        mesh=mesh,
        scratch_types=[
            pltpu.VMEM((_CH,), jnp.int32),
            pltpu.VMEM((_CH,), jnp.int32),
            pltpu.VMEM((2, _CH, 2 * D), jnp.float32),
            pltpu.VMEM((2, _CH, D), jnp.float32),
            pltpu.VMEM((2 * MAXLEN, D), jnp.float32),
            pltpu.SemaphoreType.DMA((2,)),
            pltpu.SemaphoreType.DMA((2,)),
        ],
        compiler_params=pltpu.CompilerParams(use_tc_tiling_on_sc=True),
    )


def kernel(sequence, token_table, genre_table, token_to_genres):
    fused = _build_fused(token_table, genre_table, token_to_genres)
    seqf = sequence.reshape(N)
    pe2 = jnp.asarray(np.concatenate([_PE, _PE], axis=0))
    out = _gather_pe()(fused, seqf, pe2)
    return out.reshape(BATCH, SEQLEN, D)
